# Initial kernel scaffold; baseline (speedup 1.0000x reference)
import jax
import jax.numpy as jnp
from jax.experimental import pallas as pl


def kernel(x, edge_index, W0_as, b0_as, W0_ad, b0_ad, W0_v, b0_v, W0_m, b0_m, W0_g, b0_g, W0_o, b0_o, W1_as, b1_as, W1_ad, b1_ad, W1_v, b1_v, W1_m, b1_m, W1_g, b1_g, W1_o, b1_o):
    raise NotImplementedError("write your pallas kernel here")


# speedup vs baseline: 1.0002x; 1.0002x over previous
import functools
import jax
import jax.numpy as jnp
from jax.experimental import pallas as pl
from jax.experimental.pallas import tpu as pltpu

HEADS = 8
D_A = 24
D_V = 16
D_M = 64
NEG_SLOPE = 0.1


def _mm_body(x_ref, w_ref, b_ref, o_ref):
    o_ref[...] = (
        jnp.dot(x_ref[...], w_ref[...], preferred_element_type=jnp.float32)
        + b_ref[...]
    )


def _mm(x, W, b, bm=2000):
    M, K = x.shape
    Nc = W.shape[1]
    return pl.pallas_call(
        _mm_body,
        grid=(M // bm,),
        in_specs=[
            pl.BlockSpec((bm, K), lambda i: (i, 0)),
            pl.BlockSpec((K, Nc), lambda i: (0, 0)),
            pl.BlockSpec((1, Nc), lambda i: (0, 0)),
        ],
        out_specs=pl.BlockSpec((bm, Nc), lambda i: (i, 0)),
        out_shape=jax.ShapeDtypeStruct((M, Nc), jnp.float32),
    )(x, W, b.reshape(1, Nc))


def _gaan_conv(x, src, dst, Was, bas, Wad, bad, Wv, bv, Wm, bm, Wg, bg, Wo, bo):
    N = x.shape[0]
    a_s = _mm(x, Was, bas).reshape(N, HEADS, D_A)
    a_d = _mm(x, Wad, bad).reshape(N, HEADS, D_A)
    v = _mm(x, Wv, bv).reshape(N, HEADS, D_V)
    scores = jnp.sum(a_d[dst] * a_s[src], axis=-1)
    m = jax.ops.segment_max(scores, dst, num_segments=N)
    e = jnp.exp(scores - m[dst])
    denom = jax.ops.segment_sum(e, dst, num_segments=N)
    alpha = e / (denom[dst] + 1e-16)
    agg = jax.ops.segment_sum(alpha[:, :, None] * v[src], dst, num_segments=N)
    mp = _mm(x, Wm, bm)
    maxm = jax.ops.segment_max(mp[src], dst, num_segments=N)
    deg = jax.ops.segment_sum(jnp.ones_like(src, dtype=x.dtype), dst, num_segments=N)
    maxm = jnp.where(deg[:, None] > 0, maxm, 0.0)
    meanx = jax.ops.segment_sum(x[src], dst, num_segments=N) / jnp.maximum(
        deg[:, None], 1.0
    )
    gate = jax.nn.sigmoid(_mm(jnp.concatenate([x, maxm, meanx], axis=-1), Wg, bg))
    gated = (gate[:, :, None] * agg).reshape(N, HEADS * D_V)
    return _mm(jnp.concatenate([x, gated], axis=-1), Wo, bo)


def kernel(x, edge_index,
           W0_as, b0_as, W0_ad, b0_ad, W0_v, b0_v, W0_m, b0_m, W0_g, b0_g, W0_o, b0_o,
           W1_as, b1_as, W1_ad, b1_ad, W1_v, b1_v, W1_m, b1_m, W1_g, b1_g, W1_o, b1_o):
    src = edge_index[0].astype(jnp.int32)
    dst = edge_index[1].astype(jnp.int32)
    h = _gaan_conv(x, src, dst, W0_as, b0_as, W0_ad, b0_ad, W0_v, b0_v,
                   W0_m, b0_m, W0_g, b0_g, W0_o, b0_o)
    h = jax.nn.leaky_relu(h, NEG_SLOPE)
    return _gaan_conv(h, src, dst, W1_as, b1_as, W1_ad, b1_ad, W1_v, b1_v,
                      W1_m, b1_m, W1_g, b1_g, W1_o, b1_o)


# speedup vs baseline: 9.4345x; 9.4330x over previous
import functools

import jax
import jax.numpy as jnp
from jax import lax
from jax.experimental import pallas as pl
from jax.experimental.pallas import tpu as pltpu
from jax.experimental.pallas import tpu_sc as plsc

N_NODES = 10000
N_EDGES = 320000
HEADS = 8
D_A = 24
D_V = 16
D_M = 64
NEG_SLOPE = 0.1

NW = 32
NPT = 320
CHUNK = 32
FETCH = CHUNK + 8
PW = 640
ADW = 256
OW = 320
OFFS_LEN = NW * NPT + 16


def _mm_body(x_ref, w_ref, b_ref, o_ref):
    o_ref[...] = (
        jnp.dot(x_ref[...], w_ref[...], preferred_element_type=jnp.float32)
        + b_ref[...]
    )


def _mm(x, W, b, bm=2000):
    M, K = x.shape
    Nc = W.shape[1]
    return pl.pallas_call(
        _mm_body,
        grid=(M // bm,),
        in_specs=[
            pl.BlockSpec((bm, K), lambda i: (i, 0)),
            pl.BlockSpec((K, Nc), lambda i: (0, 0)),
            pl.BlockSpec((1, Nc), lambda i: (0, 0)),
        ],
        out_specs=pl.BlockSpec((bm, Nc), lambda i: (i, 0)),
        out_shape=jax.ShapeDtypeStruct((M, Nc), jnp.float32),
    )(x, W, b.reshape(1, Nc))


def _head_pad(W, b):
    fin = W.shape[0]
    Wp = jnp.pad(W.reshape(fin, HEADS, D_A), ((0, 0), (0, 0), (0, 32 - D_A)))
    bp = jnp.pad(b.reshape(HEADS, D_A), ((0, 0), (0, 32 - D_A)))
    return Wp.reshape(fin, HEADS * 32), bp.reshape(HEADS * 32)


_GDN = lax.GatherDimensionNumbers(
    offset_dims=(), collapsed_slice_dims=(0,), start_index_map=(0,))


def _lane_perm(x, idx):
    return lax.gather(x, idx[:, None], _GDN, (1,),
                      mode=lax.GatherScatterMode.PROMISE_IN_BOUNDS)


def _lane_sum(x):
    iota = lax.iota(jnp.int32, 16)
    for sh in (8, 4, 2, 1):
        x = x + _lane_perm(x, iota ^ sh)
    return x


def _sc_edge_kernel(p_hbm, ad_hbm, ssrc_hbm, offs_hbm, out_hbm,
                    offs_v, idx_v, rows_v, adrow_v, ob_v, sem):
    cid = lax.axis_index("c")
    sid = lax.axis_index("s")
    wid = sid * 2 + cid
    n0 = wid * NPT

    pltpu.sync_copy(offs_hbm.at[pl.ds(n0, NPT + 16)], offs_v)

    n_here = jnp.maximum(jnp.minimum(NPT, N_NODES - n0), 0)

    def node_body(i, _):
        st = offs_v[pl.ds(i, 16)]
        s = st[0]
        t = st[1]
        deg = t - s
        degf = jnp.broadcast_to(deg.astype(jnp.float32), (16,))

        pltpu.sync_copy(ad_hbm.at[n0 + i], adrow_v)
        ad = [adrow_v[pl.ds(16 * j, 16)] for j in range(16)]

        zero = jnp.zeros((16,), jnp.float32)
        wv = [zero] * HEADS
        dens = [zero] * HEADS
        sx = [zero] * 8
        mx = [jnp.full((16,), -3.0e38, jnp.float32)] * 4

        nchunks = (deg + (CHUNK - 1)) // CHUNK

        def chunk_body(j, carry):
            wv, dens, sx, mx = carry
            e0 = s + j * CHUNK
            a = (e0 // 8) * 8
            sh = e0 - a
            cnt = jnp.minimum(CHUNK, t - e0)
            pltpu.sync_copy(ssrc_hbm.at[pl.ds(a, FETCH)], idx_v)
            pltpu.async_copy(p_hbm.at[idx_v], rows_v, sem).wait()

            def edge_body(e, carry):
                wv, dens, sx, mx = carry
                r = sh + e
                new_wv = []
                new_dens = []
                for h in range(HEADS):
                    p0 = rows_v[r, pl.ds(32 * h, 16)] * ad[2 * h]
                    p1 = rows_v[r, pl.ds(32 * h + 16, 16)] * ad[2 * h + 1]
                    es = jnp.exp(_lane_sum(p0 + p1))
                    vv = rows_v[r, pl.ds(256 + 16 * h, 16)]
                    new_wv.append(wv[h] + es * vv)
                    new_dens.append(dens[h] + es)
                new_sx = [sx[k] + rows_v[r, pl.ds(384 + 16 * k, 16)]
                          for k in range(8)]
                new_mx = [jnp.maximum(mx[k], rows_v[r, pl.ds(512 + 16 * k, 16)])
                          for k in range(4)]
                return (tuple(new_wv), tuple(new_dens), tuple(new_sx),
                        tuple(new_mx))

            return lax.fori_loop(0, cnt, edge_body, (wv, dens, sx, mx))

        wv, dens, sx, mx = lax.fori_loop(
            0, nchunks, chunk_body,
            (tuple(wv), tuple(dens), tuple(sx), tuple(mx)))

        for h in range(HEADS):
            ob_v[pl.ds(16 * h, 16)] = wv[h] / (dens[h] + 1e-16)
        ind = jnp.minimum(degf, 1.0)
        for k in range(4):
            ob_v[pl.ds(128 + 16 * k, 16)] = mx[k] * ind
        inv_deg = 1.0 / jnp.maximum(degf, 1.0)
        for k in range(8):
            ob_v[pl.ds(192 + 16 * k, 16)] = sx[k] * inv_deg
        pltpu.sync_copy(ob_v, out_hbm.at[n0 + i])
        return 0

    lax.fori_loop(0, n_here, node_body, 0)


def _sc_edge_phase(P, AD, ssrc_pad, offs_pad):
    mesh = plsc.VectorSubcoreMesh(core_axis_name="c", subcore_axis_name="s")
    k = functools.partial(
        pl.kernel,
        mesh=mesh,
        out_type=jax.ShapeDtypeStruct((N_NODES, OW), jnp.float32),
        scratch_types=[
            pltpu.VMEM((NPT + 16,), jnp.int32),
            pltpu.VMEM((FETCH,), jnp.int32),
            pltpu.VMEM((FETCH, PW), jnp.float32),
            pltpu.VMEM((ADW,), jnp.float32),
            pltpu.VMEM((OW,), jnp.float32),
            pltpu.SemaphoreType.DMA,
        ],
    )(_sc_edge_kernel)
    return k(P, AD, ssrc_pad, offs_pad)


def _epilogue_body(relu, x_ref, o_ref, wgx_ref, wgm_ref, bg_ref,
                   wot_ref, wob_ref, bo_ref, out_ref):
    x = x_ref[...]
    gi = (jnp.dot(x, wgx_ref[...], preferred_element_type=jnp.float32)
          + jnp.dot(o_ref[:, 128:320], wgm_ref[...],
                    preferred_element_type=jnp.float32)
          + bg_ref[...])
    gate = jax.nn.sigmoid(gi)
    bm = x.shape[0]
    agg = o_ref[:, 0:128].reshape(bm, HEADS, D_V)
    gated = (gate[:, :, None] * agg).reshape(bm, HEADS * D_V)
    out = (jnp.dot(x, wot_ref[...], preferred_element_type=jnp.float32)
           + jnp.dot(gated, wob_ref[...], preferred_element_type=jnp.float32)
           + bo_ref[...])
    if relu:
        out = jnp.where(out >= 0.0, out, NEG_SLOPE * out)
    out_ref[...] = out


def _epilogue(x, OUT, Wg, bg, Wo, bo, relu, bm=2000):
    M, F = x.shape
    fout = Wo.shape[1]
    Wgx = Wg[:F]
    Wgm = Wg[F:]
    Wot = Wo[:F]
    Wob = Wo[F:]
    return pl.pallas_call(
        functools.partial(_epilogue_body, relu),
        grid=(M // bm,),
        in_specs=[
            pl.BlockSpec((bm, F), lambda i: (i, 0)),
            pl.BlockSpec((bm, OW), lambda i: (i, 0)),
            pl.BlockSpec((F, HEADS), lambda i: (0, 0)),
            pl.BlockSpec((192, HEADS), lambda i: (0, 0)),
            pl.BlockSpec((1, HEADS), lambda i: (0, 0)),
            pl.BlockSpec((F, fout), lambda i: (0, 0)),
            pl.BlockSpec((128, fout), lambda i: (0, 0)),
            pl.BlockSpec((1, fout), lambda i: (0, 0)),
        ],
        out_specs=pl.BlockSpec((bm, fout), lambda i: (i, 0)),
        out_shape=jax.ShapeDtypeStruct((M, fout), jnp.float32),
    )(x, OUT, Wgx, Wgm, bg.reshape(1, HEADS), Wot, Wob, bo.reshape(1, fout))


def _gaan_layer(x, ssrc_pad, offs_pad,
                Was, bas, Wad, bad, Wv, bv, Wm, bm_, Wg, bg, Wo, bo, relu):
    F = x.shape[1]
    Wasp, basp = _head_pad(Was, bas)
    Wadp, badp = _head_pad(Wad, bad)
    eye = jnp.eye(F, dtype=jnp.float32)
    Wcat = jnp.concatenate(
        [Wasp, Wv, eye, Wm, jnp.zeros((F, 64), jnp.float32)], axis=1)
    bcat = jnp.concatenate(
        [basp, bv, jnp.zeros((F,), jnp.float32), bm_,
         jnp.zeros((64,), jnp.float32)], axis=0)
    P = _mm(x, Wcat, bcat)
    AD = _mm(x, Wadp, badp)
    OUT = _sc_edge_phase(P, AD, ssrc_pad, offs_pad)
    return _epilogue(x, OUT, Wg, bg, Wo, bo, relu)


def kernel(x, edge_index,
           W0_as, b0_as, W0_ad, b0_ad, W0_v, b0_v, W0_m, b0_m, W0_g, b0_g, W0_o, b0_o,
           W1_as, b1_as, W1_ad, b1_ad, W1_v, b1_v, W1_m, b1_m, W1_g, b1_g, W1_o, b1_o):
    src = edge_index[0].astype(jnp.int32)
    dst = edge_index[1].astype(jnp.int32)
    order = jnp.argsort(dst)
    ssrc = src[order]
    sdst = dst[order]
    offs = jnp.searchsorted(sdst, jnp.arange(N_NODES + 1, dtype=jnp.int32)
                            ).astype(jnp.int32)
    ssrc_pad = jnp.concatenate(
        [ssrc, jnp.zeros((FETCH,), jnp.int32)], axis=0)
    offs_pad = jnp.concatenate(
        [offs, jnp.full((OFFS_LEN - (N_NODES + 1),), N_EDGES, jnp.int32)],
        axis=0)

    h = _gaan_layer(x, ssrc_pad, offs_pad,
                    W0_as, b0_as, W0_ad, b0_ad, W0_v, b0_v, W0_m, b0_m,
                    W0_g, b0_g, W0_o, b0_o, relu=True)
    return _gaan_layer(h, ssrc_pad, offs_pad,
                       W1_as, b1_as, W1_ad, b1_ad, W1_v, b1_v, W1_m, b1_m,
                       W1_g, b1_g, W1_o, b1_o, relu=False)


# speedup vs baseline: 17.7852x; 1.8851x over previous
import functools

import jax
import jax.numpy as jnp
from jax import lax
from jax.experimental import pallas as pl
from jax.experimental.pallas import tpu as pltpu
from jax.experimental.pallas import tpu_sc as plsc

N_NODES = 10000
N_EDGES = 320000
HEADS = 8
D_A = 24
D_V = 16
D_M = 64
NEG_SLOPE = 0.1

NW = 32
NPT = 320
CHUNK = 32
FETCH = CHUNK + 8
PW = 640
ADW = 256
OW = 320
OFFS_LEN = NW * NPT + 16


def _mm_body(x_ref, w_ref, b_ref, o_ref):
    o_ref[...] = (
        jnp.dot(x_ref[...], w_ref[...], preferred_element_type=jnp.float32)
        + b_ref[...]
    )


def _mm(x, W, b, bm=2000):
    M, K = x.shape
    Nc = W.shape[1]
    return pl.pallas_call(
        _mm_body,
        grid=(M // bm,),
        in_specs=[
            pl.BlockSpec((bm, K), lambda i: (i, 0)),
            pl.BlockSpec((K, Nc), lambda i: (0, 0)),
            pl.BlockSpec((1, Nc), lambda i: (0, 0)),
        ],
        out_specs=pl.BlockSpec((bm, Nc), lambda i: (i, 0)),
        out_shape=jax.ShapeDtypeStruct((M, Nc), jnp.float32),
    )(x, W, b.reshape(1, Nc))


def _head_pad(W, b):
    fin = W.shape[0]
    Wp = jnp.pad(W.reshape(fin, HEADS, D_A), ((0, 0), (0, 0), (0, 32 - D_A)))
    bp = jnp.pad(b.reshape(HEADS, D_A), ((0, 0), (0, 32 - D_A)))
    return Wp.reshape(fin, HEADS * 32), bp.reshape(HEADS * 32)


_GDN = lax.GatherDimensionNumbers(
    offset_dims=(), collapsed_slice_dims=(0,), start_index_map=(0,))


def _lane_perm(x, idx):
    return lax.gather(x, idx[:, None], _GDN, (1,),
                      mode=lax.GatherScatterMode.PROMISE_IN_BOUNDS)


def _lane_sum(x):
    iota = lax.iota(jnp.int32, 16)
    for sh in (8, 4, 2, 1):
        x = x + _lane_perm(x, iota ^ sh)
    return x


def _sc_edge_kernel(p_hbm, ad_hbm, ssrc_hbm, offs_hbm, out_hbm,
                    offs_v, idx_v, rows_v, adrow_v, ob_v, sem):
    cid = lax.axis_index("c")
    sid = lax.axis_index("s")
    wid = sid * 2 + cid
    n0 = wid * NPT

    pltpu.sync_copy(offs_hbm.at[pl.ds(n0, NPT + 16)], offs_v)

    n_here = jnp.maximum(jnp.minimum(NPT, N_NODES - n0), 0)

    def node_body(i, _):
        st = offs_v[pl.ds(i, 16)]
        s = st[0]
        t = st[1]
        deg = t - s
        degf = jnp.broadcast_to(deg.astype(jnp.float32), (16,))

        pltpu.sync_copy(ad_hbm.at[n0 + i], adrow_v)
        ad = [adrow_v[pl.ds(16 * j, 16)] for j in range(16)]

        zero = jnp.zeros((16,), jnp.float32)
        wv = [zero] * HEADS
        dens = [zero] * HEADS
        sx = [zero] * 8
        mx = [jnp.full((16,), -3.0e38, jnp.float32)] * 4

        nchunks = (deg + (CHUNK - 1)) // CHUNK

        def chunk_body(j, carry):
            wv, dens, sx, mx = carry
            e0 = s + j * CHUNK
            a = (e0 // 8) * 8
            sh = e0 - a
            cnt = jnp.minimum(CHUNK, t - e0)
            pltpu.sync_copy(ssrc_hbm.at[pl.ds(a, FETCH)], idx_v)
            pltpu.async_copy(p_hbm.at[idx_v], rows_v, sem).wait()

            def edge_body(e, carry):
                wv, dens, sx, mx = carry
                r = sh + e
                new_wv = []
                new_dens = []
                for h in range(HEADS):
                    p0 = rows_v[r, pl.ds(32 * h, 16)] * ad[2 * h]
                    p1 = rows_v[r, pl.ds(32 * h + 16, 16)] * ad[2 * h + 1]
                    es = jnp.exp(_lane_sum(p0 + p1))
                    vv = rows_v[r, pl.ds(256 + 16 * h, 16)]
                    new_wv.append(wv[h] + es * vv)
                    new_dens.append(dens[h] + es)
                new_sx = [sx[k] + rows_v[r, pl.ds(384 + 16 * k, 16)]
                          for k in range(8)]
                new_mx = [jnp.maximum(mx[k], rows_v[r, pl.ds(512 + 16 * k, 16)])
                          for k in range(4)]
                return (tuple(new_wv), tuple(new_dens), tuple(new_sx),
                        tuple(new_mx))

            return lax.fori_loop(0, cnt, edge_body, (wv, dens, sx, mx))

        wv, dens, sx, mx = lax.fori_loop(
            0, nchunks, chunk_body,
            (tuple(wv), tuple(dens), tuple(sx), tuple(mx)))

        for h in range(HEADS):
            ob_v[pl.ds(16 * h, 16)] = wv[h] / (dens[h] + 1e-16)
        ind = jnp.minimum(degf, 1.0)
        for k in range(4):
            ob_v[pl.ds(128 + 16 * k, 16)] = mx[k] * ind
        inv_deg = 1.0 / jnp.maximum(degf, 1.0)
        for k in range(8):
            ob_v[pl.ds(192 + 16 * k, 16)] = sx[k] * inv_deg
        pltpu.sync_copy(ob_v, out_hbm.at[n0 + i])
        return 0

    lax.fori_loop(0, n_here, node_body, 0)


def _sc_edge_phase(P, AD, ssrc_pad, offs_pad):
    mesh = plsc.VectorSubcoreMesh(core_axis_name="c", subcore_axis_name="s")
    k = functools.partial(
        pl.kernel,
        mesh=mesh,
        out_type=jax.ShapeDtypeStruct((N_NODES, OW), jnp.float32),
        scratch_types=[
            pltpu.VMEM((NPT + 16,), jnp.int32),
            pltpu.VMEM((FETCH,), jnp.int32),
            pltpu.VMEM((FETCH, PW), jnp.float32),
            pltpu.VMEM((ADW,), jnp.float32),
            pltpu.VMEM((OW,), jnp.float32),
            pltpu.SemaphoreType.DMA,
        ],
    )(_sc_edge_kernel)
    return k(P, AD, ssrc_pad, offs_pad)


def _epilogue_body(relu, x_ref, o_ref, wgx_ref, wgm_ref, bg_ref,
                   wot_ref, wob_ref, bo_ref, out_ref):
    x = x_ref[...]
    gi = (jnp.dot(x, wgx_ref[...], preferred_element_type=jnp.float32)
          + jnp.dot(o_ref[:, 128:320], wgm_ref[...],
                    preferred_element_type=jnp.float32)
          + bg_ref[...])
    gate = jax.nn.sigmoid(gi)
    bm = x.shape[0]
    agg = o_ref[:, 0:128].reshape(bm, HEADS, D_V)
    gated = (gate[:, :, None] * agg).reshape(bm, HEADS * D_V)
    out = (jnp.dot(x, wot_ref[...], preferred_element_type=jnp.float32)
           + jnp.dot(gated, wob_ref[...], preferred_element_type=jnp.float32)
           + bo_ref[...])
    if relu:
        out = jnp.where(out >= 0.0, out, NEG_SLOPE * out)
    out_ref[...] = out


def _epilogue(x, OUT, Wg, bg, Wo, bo, relu, bm=2000):
    M, F = x.shape
    fout = Wo.shape[1]
    Wgx = Wg[:F]
    Wgm = Wg[F:]
    Wot = Wo[:F]
    Wob = Wo[F:]
    return pl.pallas_call(
        functools.partial(_epilogue_body, relu),
        grid=(M // bm,),
        in_specs=[
            pl.BlockSpec((bm, F), lambda i: (i, 0)),
            pl.BlockSpec((bm, OW), lambda i: (i, 0)),
            pl.BlockSpec((F, HEADS), lambda i: (0, 0)),
            pl.BlockSpec((192, HEADS), lambda i: (0, 0)),
            pl.BlockSpec((1, HEADS), lambda i: (0, 0)),
            pl.BlockSpec((F, fout), lambda i: (0, 0)),
            pl.BlockSpec((128, fout), lambda i: (0, 0)),
            pl.BlockSpec((1, fout), lambda i: (0, 0)),
        ],
        out_specs=pl.BlockSpec((bm, fout), lambda i: (i, 0)),
        out_shape=jax.ShapeDtypeStruct((M, fout), jnp.float32),
    )(x, OUT, Wgx, Wgm, bg.reshape(1, HEADS), Wot, Wob, bo.reshape(1, fout))


def _gaan_layer(x, ssrc_pad, offs_pad,
                Was, bas, Wad, bad, Wv, bv, Wm, bm_, Wg, bg, Wo, bo, relu):
    F = x.shape[1]
    Wasp, basp = _head_pad(Was, bas)
    Wadp, badp = _head_pad(Wad, bad)
    eye = jnp.eye(F, dtype=jnp.float32)
    Wcat = jnp.concatenate(
        [Wasp, Wv, eye, Wm, jnp.zeros((F, 64), jnp.float32)], axis=1)
    bcat = jnp.concatenate(
        [basp, bv, jnp.zeros((F,), jnp.float32), bm_,
         jnp.zeros((64,), jnp.float32)], axis=0)
    P = _mm(x, Wcat, bcat)
    AD = _mm(x, Wadp, badp)
    OUT = _sc_edge_phase(P, AD, ssrc_pad, offs_pad)
    return _epilogue(x, OUT, Wg, bg, Wo, bo, relu)


def kernel(x, edge_index,
           W0_as, b0_as, W0_ad, b0_ad, W0_v, b0_v, W0_m, b0_m, W0_g, b0_g, W0_o, b0_o,
           W1_as, b1_as, W1_ad, b1_ad, W1_v, b1_v, W1_m, b1_m, W1_g, b1_g, W1_o, b1_o):
    src = edge_index[0].astype(jnp.int32)
    dst = edge_index[1].astype(jnp.int32)
    packed = jnp.sort(dst * 16384 + src)
    ssrc = packed & 16383
    sdst = packed >> 14
    offs = jnp.searchsorted(sdst, jnp.arange(N_NODES + 1, dtype=jnp.int32)
                            ).astype(jnp.int32)
    ssrc_pad = jnp.concatenate(
        [ssrc, jnp.zeros((FETCH,), jnp.int32)], axis=0)
    offs_pad = jnp.concatenate(
        [offs, jnp.full((OFFS_LEN - (N_NODES + 1),), N_EDGES, jnp.int32)],
        axis=0)

    h = _gaan_layer(x, ssrc_pad, offs_pad,
                    W0_as, b0_as, W0_ad, b0_ad, W0_v, b0_v, W0_m, b0_m,
                    W0_g, b0_g, W0_o, b0_o, relu=True)
    return _gaan_layer(h, ssrc_pad, offs_pad,
                       W1_as, b1_as, W1_ad, b1_ad, W1_v, b1_v, W1_m, b1_m,
                       W1_g, b1_g, W1_o, b1_o, relu=False)


# speedup vs baseline: 29.6393x; 1.6665x over previous
import functools

import jax
import jax.numpy as jnp
from jax import lax
from jax.experimental import pallas as pl
from jax.experimental.pallas import tpu as pltpu
from jax.experimental.pallas import tpu_sc as plsc

N_NODES = 10000
N_EDGES = 320000
HEADS = 8
D_A = 24
D_V = 16
D_M = 64
NEG_SLOPE = 0.1

NW = 32
NPT = 320
CHUNK = 40
FETCH = CHUNK + 8
RING = 4096
PW = 640
ADW = 256
OW = 320
OFFS_LEN = NW * NPT + 16


def _mm_body(x_ref, w_ref, b_ref, o_ref):
    o_ref[...] = (
        jnp.dot(x_ref[...], w_ref[...], preferred_element_type=jnp.float32)
        + b_ref[...]
    )


def _mm(x, W, b, bm=2000):
    M, K = x.shape
    Nc = W.shape[1]
    return pl.pallas_call(
        _mm_body,
        grid=(M // bm,),
        in_specs=[
            pl.BlockSpec((bm, K), lambda i: (i, 0)),
            pl.BlockSpec((K, Nc), lambda i: (0, 0)),
            pl.BlockSpec((1, Nc), lambda i: (0, 0)),
        ],
        out_specs=pl.BlockSpec((bm, Nc), lambda i: (i, 0)),
        out_shape=jax.ShapeDtypeStruct((M, Nc), jnp.float32),
    )(x, W, b.reshape(1, Nc))


def _head_pad(W, b):
    fin = W.shape[0]
    Wp = jnp.pad(W.reshape(fin, HEADS, D_A), ((0, 0), (0, 0), (0, 32 - D_A)))
    bp = jnp.pad(b.reshape(HEADS, D_A), ((0, 0), (0, 32 - D_A)))
    return Wp.reshape(fin, HEADS * 32), bp.reshape(HEADS * 32)


_GDN = lax.GatherDimensionNumbers(
    offset_dims=(), collapsed_slice_dims=(0,), start_index_map=(0,))


def _lane_perm(x, idx):
    return lax.gather(x, idx[:, None], _GDN, (1,),
                      mode=lax.GatherScatterMode.PROMISE_IN_BOUNDS)


def _lane_sum(x):
    iota = lax.iota(jnp.int32, 16)
    for sh in (8, 4, 2, 1):
        x = x + _lane_perm(x, iota ^ sh)
    return x


def _f8(v):
    return (v // 8) * 8


def _sc_edge_kernel(p_hbm, ad_hbm, ssrc_hbm, offs_hbm, out_hbm,
                    offs_v, ring_v, rows_v, ov_v, xidx_v, adb_v, ob_v,
                    gsem, asem, osem, xsem):
    cid = lax.axis_index("c")
    sid = lax.axis_index("s")
    wid = sid * 2 + cid
    n0 = wid * NPT

    pltpu.sync_copy(offs_hbm.at[pl.ds(n0, NPT + 16)], offs_v)
    n_here = jnp.minimum(NPT, N_NODES - n0)

    def issue_node(i, par, rb):
        st = offs_v[pl.ds(i, 16)]
        a = _f8(st[0])
        need = a + FETCH > rb + RING
        rb_new = jnp.where(need, a, rb)

        @pl.when(need)
        def _():
            pltpu.sync_copy(ssrc_hbm.at[pl.ds(a, RING)], ring_v)

        off = pl.multiple_of(a - rb_new, 8)
        pltpu.async_copy(p_hbm.at[ring_v.at[pl.ds(off, FETCH)]],
                         rows_v.at[par], gsem.at[par])
        pltpu.async_copy(ad_hbm.at[n0 + i], adb_v.at[par], asem.at[par])
        return rb_new

    def make_edge_body(rowfn, ad):
        def edge_body(e, carry):
            wv, dens, sx, mx = carry
            new_wv = []
            new_dens = []
            for h in range(HEADS):
                p0 = rowfn(e, 32 * h) * ad[2 * h]
                p1 = rowfn(e, 32 * h + 16) * ad[2 * h + 1]
                es = jnp.exp(_lane_sum(p0 + p1))
                vv = rowfn(e, 256 + 16 * h)
                new_wv.append(wv[h] + es * vv)
                new_dens.append(dens[h] + es)
            new_sx = tuple(sx[k] + rowfn(e, 384 + 16 * k) for k in range(8))
            new_mx = tuple(jnp.maximum(mx[k], rowfn(e, 512 + 16 * k))
                           for k in range(4))
            return (tuple(new_wv), tuple(new_dens), new_sx, new_mx)
        return edge_body

    rb0 = issue_node(0, 0, jnp.int32(-RING))

    def node_body(i, rb):
        par = i & 1
        pltpu.make_async_copy(p_hbm.at[pl.ds(0, FETCH)], rows_v.at[par],
                              gsem.at[par]).wait()
        pltpu.make_async_copy(ad_hbm.at[0], adb_v.at[par],
                              asem.at[par]).wait()
        rb_new = lax.cond(i + 1 < n_here,
                          lambda r: issue_node(i + 1, 1 - par, r),
                          lambda r: r, rb)

        st = offs_v[pl.ds(i, 16)]
        s = st[0]
        t = st[1]
        deg = t - s
        degf = jnp.broadcast_to(deg.astype(jnp.float32), (16,))
        sh = s - _f8(s)
        ad = [adb_v[par, pl.ds(16 * j, 16)] for j in range(16)]

        zero = jnp.zeros((16,), jnp.float32)
        init = (tuple([zero] * 8), tuple([zero] * 8), tuple([zero] * 8),
                tuple([jnp.full((16,), -3.0e38, jnp.float32)] * 4))

        def rowfn0(e, off):
            return rows_v[par, sh + e, pl.ds(off, 16)]

        acc = lax.fori_loop(0, jnp.minimum(deg, CHUNK),
                            make_edge_body(rowfn0, ad), init)

        nov = (jnp.maximum(deg - CHUNK, 0) + CHUNK - 1) // CHUNK

        def ov_body(j, acc):
            e0 = s + CHUNK * (j + 1)
            a = _f8(e0)
            shj = e0 - a
            cntj = jnp.minimum(CHUNK, t - e0)
            pltpu.sync_copy(ssrc_hbm.at[pl.ds(a, FETCH)], xidx_v)
            pltpu.async_copy(p_hbm.at[xidx_v], ov_v, xsem).wait()

            def rowfnj(e, off):
                return ov_v[shj + e, pl.ds(off, 16)]

            return lax.fori_loop(0, cntj, make_edge_body(rowfnj, ad), acc)

        wv, dens, sx, mx = lax.fori_loop(0, nov, ov_body, acc)

        @pl.when(i >= 2)
        def _():
            pltpu.make_async_copy(ob_v.at[par], out_hbm.at[0],
                                  osem.at[par]).wait()
        for h in range(HEADS):
            ob_v[par, pl.ds(16 * h, 16)] = wv[h] / (dens[h] + 1e-16)
        ind = jnp.minimum(degf, 1.0)
        for k in range(4):
            ob_v[par, pl.ds(128 + 16 * k, 16)] = mx[k] * ind
        inv_deg = 1.0 / jnp.maximum(degf, 1.0)
        for k in range(8):
            ob_v[par, pl.ds(192 + 16 * k, 16)] = sx[k] * inv_deg
        pltpu.async_copy(ob_v.at[par], out_hbm.at[n0 + i], osem.at[par])
        return rb_new

    lax.fori_loop(0, n_here, node_body, rb0)

    @pl.when(n_here >= 1)
    def _():
        pltpu.make_async_copy(ob_v.at[0], out_hbm.at[0],
                              osem.at[(n_here - 1) & 1]).wait()

    @pl.when(n_here >= 2)
    def _():
        pltpu.make_async_copy(ob_v.at[0], out_hbm.at[0],
                              osem.at[(n_here - 2) & 1]).wait()


def _sc_edge_phase(P, AD, ssrc_pad, offs_pad):
    mesh = plsc.VectorSubcoreMesh(core_axis_name="c", subcore_axis_name="s")
    k = functools.partial(
        pl.kernel,
        mesh=mesh,
        out_type=jax.ShapeDtypeStruct((N_NODES, OW), jnp.float32),
        scratch_types=[
            pltpu.VMEM((NPT + 16,), jnp.int32),
            pltpu.VMEM((RING,), jnp.int32),
            pltpu.VMEM((2, FETCH, PW), jnp.float32),
            pltpu.VMEM((FETCH, PW), jnp.float32),
            pltpu.VMEM((FETCH,), jnp.int32),
            pltpu.VMEM((2, ADW), jnp.float32),
            pltpu.VMEM((2, OW), jnp.float32),
            pltpu.SemaphoreType.DMA((2,)),
            pltpu.SemaphoreType.DMA((2,)),
            pltpu.SemaphoreType.DMA((2,)),
            pltpu.SemaphoreType.DMA,
        ],
    )(_sc_edge_kernel)
    return k(P, AD, ssrc_pad, offs_pad)


def _epilogue_body(relu, x_ref, o_ref, wgx_ref, wgm_ref, bg_ref,
                   wot_ref, wob_ref, bo_ref, out_ref):
    x = x_ref[...]
    gi = (jnp.dot(x, wgx_ref[...], preferred_element_type=jnp.float32)
          + jnp.dot(o_ref[:, 128:320], wgm_ref[...],
                    preferred_element_type=jnp.float32)
          + bg_ref[...])
    gate = jax.nn.sigmoid(gi)
    bm = x.shape[0]
    agg = o_ref[:, 0:128].reshape(bm, HEADS, D_V)
    gated = (gate[:, :, None] * agg).reshape(bm, HEADS * D_V)
    out = (jnp.dot(x, wot_ref[...], preferred_element_type=jnp.float32)
           + jnp.dot(gated, wob_ref[...], preferred_element_type=jnp.float32)
           + bo_ref[...])
    if relu:
        out = jnp.where(out >= 0.0, out, NEG_SLOPE * out)
    out_ref[...] = out


def _epilogue(x, OUT, Wg, bg, Wo, bo, relu, bm=2000):
    M, F = x.shape
    fout = Wo.shape[1]
    Wgx = Wg[:F]
    Wgm = Wg[F:]
    Wot = Wo[:F]
    Wob = Wo[F:]
    return pl.pallas_call(
        functools.partial(_epilogue_body, relu),
        grid=(M // bm,),
        in_specs=[
            pl.BlockSpec((bm, F), lambda i: (i, 0)),
            pl.BlockSpec((bm, OW), lambda i: (i, 0)),
            pl.BlockSpec((F, HEADS), lambda i: (0, 0)),
            pl.BlockSpec((192, HEADS), lambda i: (0, 0)),
            pl.BlockSpec((1, HEADS), lambda i: (0, 0)),
            pl.BlockSpec((F, fout), lambda i: (0, 0)),
            pl.BlockSpec((128, fout), lambda i: (0, 0)),
            pl.BlockSpec((1, fout), lambda i: (0, 0)),
        ],
        out_specs=pl.BlockSpec((bm, fout), lambda i: (i, 0)),
        out_shape=jax.ShapeDtypeStruct((M, fout), jnp.float32),
    )(x, OUT, Wgx, Wgm, bg.reshape(1, HEADS), Wot, Wob, bo.reshape(1, fout))


def _gaan_layer(x, ssrc_pad, offs_pad,
                Was, bas, Wad, bad, Wv, bv, Wm, bm_, Wg, bg, Wo, bo, relu):
    F = x.shape[1]
    Wasp, basp = _head_pad(Was, bas)
    Wadp, badp = _head_pad(Wad, bad)
    eye = jnp.eye(F, dtype=jnp.float32)
    Wcat = jnp.concatenate(
        [Wasp, Wv, eye, Wm, jnp.zeros((F, 64), jnp.float32)], axis=1)
    bcat = jnp.concatenate(
        [basp, bv, jnp.zeros((F,), jnp.float32), bm_,
         jnp.zeros((64,), jnp.float32)], axis=0)
    P = _mm(x, Wcat, bcat)
    AD = _mm(x, Wadp, badp)
    OUT = _sc_edge_phase(P, AD, ssrc_pad, offs_pad)
    return _epilogue(x, OUT, Wg, bg, Wo, bo, relu)


def kernel(x, edge_index,
           W0_as, b0_as, W0_ad, b0_ad, W0_v, b0_v, W0_m, b0_m, W0_g, b0_g, W0_o, b0_o,
           W1_as, b1_as, W1_ad, b1_ad, W1_v, b1_v, W1_m, b1_m, W1_g, b1_g, W1_o, b1_o):
    src = edge_index[0].astype(jnp.int32)
    dst = edge_index[1].astype(jnp.int32)
    packed = jnp.sort(dst * 16384 + src)
    ssrc = packed & 16383
    sdst = packed >> 14
    offs = jnp.searchsorted(sdst, jnp.arange(N_NODES + 1, dtype=jnp.int32)
                            ).astype(jnp.int32)
    ssrc_pad = jnp.concatenate(
        [ssrc, jnp.zeros((RING,), jnp.int32)], axis=0)
    offs_pad = jnp.concatenate(
        [offs, jnp.full((OFFS_LEN - (N_NODES + 1),), N_EDGES, jnp.int32)],
        axis=0)

    h = _gaan_layer(x, ssrc_pad, offs_pad,
                    W0_as, b0_as, W0_ad, b0_ad, W0_v, b0_v, W0_m, b0_m,
                    W0_g, b0_g, W0_o, b0_o, relu=True)
    return _gaan_layer(h, ssrc_pad, offs_pad,
                       W1_as, b1_as, W1_ad, b1_ad, W1_v, b1_v, W1_m, b1_m,
                       W1_g, b1_g, W1_o, b1_o, relu=False)
